# baseline (speedup 1.0000x reference)
import jax
import jax.numpy as jnp
from jax import lax
from jax.experimental import pallas as pl
from jax.experimental.pallas import tpu as pltpu
from jax.experimental.pallas import tpu_sc as plsc

N_NODES = 100000
M_EDGES = 6400000
NUM_WORKERS = 32
PER_WORKER = M_EDGES // NUM_WORKERS
CHUNK = 4000
CHUNKS = PER_WORKER // CHUNK
L = 16


def _newton_sqrt(sq):
    i = plsc.bitcast(sq, jnp.int32)
    i = jnp.int32(0x5F3759DF) - (i >> 1)
    y = plsc.bitcast(i, jnp.float32)
    t = sq * jnp.float32(0.5)
    y = y * (jnp.float32(1.5) - (t * y) * y)
    y = y * (jnp.float32(1.5) - (t * y) * y)
    y = y * (jnp.float32(1.5) - (t * y) * y)
    return sq * y


def _body(pos8_hbm, ei_hbm, eiout_hbm, w_hbm, vec_hbm,
          idx0_v, idx1_v, rows0_v, rows1_v, vec_v, w_v, sem0, sem1):
    wid = lax.axis_index("c") * 16 + lax.axis_index("s")
    base_w = wid * PER_WORKER

    iota = lax.iota(jnp.int32, L)
    c0 = jnp.zeros((L,), jnp.int32)
    c1 = jnp.full((L,), 1, jnp.int32)
    c2 = jnp.full((L,), 2, jnp.int32)

    def chunk_body(g, carry):
        base = base_w + g * CHUNK
        pltpu.sync_copy(ei_hbm.at[pl.ds(base, CHUNK)], idx0_v)
        pltpu.sync_copy(ei_hbm.at[pl.ds(M_EDGES + base, CHUNK)], idx1_v)
        cp0 = pltpu.async_copy(pos8_hbm.at[idx0_v], rows0_v, sem0)
        cp1 = pltpu.async_copy(pos8_hbm.at[idx1_v], rows1_v, sem1)
        cp0.wait()
        cp1.wait()

        def compute(i, carry2):
            ev = i * L + iota
            x0 = plsc.load_gather(rows0_v, [ev, c0])
            y0 = plsc.load_gather(rows0_v, [ev, c1])
            z0 = plsc.load_gather(rows0_v, [ev, c2])
            x1 = plsc.load_gather(rows1_v, [ev, c0])
            y1 = plsc.load_gather(rows1_v, [ev, c1])
            z1 = plsc.load_gather(rows1_v, [ev, c2])
            dx = x0 - x1
            dy = y0 - y1
            dz = z0 - z1
            sq = dx * dx + dy * dy + dz * dz
            w = jnp.maximum(_newton_sqrt(sq), jnp.float32(1e-8))
            w_v[pl.ds(i * L, L)] = w
            plsc.store_scatter(vec_v, [ev, c0], dx)
            plsc.store_scatter(vec_v, [ev, c1], dy)
            plsc.store_scatter(vec_v, [ev, c2], dz)
            return carry2

        lax.fori_loop(0, CHUNK // L, compute, 0, unroll=False)

        pltpu.sync_copy(idx0_v, eiout_hbm.at[pl.ds(base, CHUNK)])
        pltpu.sync_copy(idx1_v, eiout_hbm.at[pl.ds(M_EDGES + base, CHUNK)])
        pltpu.sync_copy(w_v, w_hbm.at[pl.ds(base, CHUNK)])
        pltpu.sync_copy(vec_v, vec_hbm.at[pl.ds(base, CHUNK)])
        return carry

    lax.fori_loop(0, CHUNKS, chunk_body, 0, unroll=False)


def _distance_sc(pos8, ei_flat):
    mesh = plsc.VectorSubcoreMesh(core_axis_name="c", subcore_axis_name="s")
    k = pl.kernel(
        _body,
        out_type=(
            jax.ShapeDtypeStruct((2 * M_EDGES,), jnp.int32),
            jax.ShapeDtypeStruct((M_EDGES,), jnp.float32),
            jax.ShapeDtypeStruct((M_EDGES, 3), jnp.float32),
        ),
        mesh=mesh,
        compiler_params=pltpu.CompilerParams(use_tc_tiling_on_sc=False,
                                             needs_layout_passes=False),
        scratch_types=[
            pltpu.VMEM((CHUNK,), jnp.int32),
            pltpu.VMEM((CHUNK,), jnp.int32),
            pltpu.VMEM((CHUNK, 8), jnp.float32),
            pltpu.VMEM((CHUNK, 8), jnp.float32),
            pltpu.VMEM((CHUNK, 3), jnp.float32),
            pltpu.VMEM((CHUNK,), jnp.float32),
            pltpu.SemaphoreType.DMA,
            pltpu.SemaphoreType.DMA,
        ],
    )
    return k(pos8, ei_flat)


def kernel(pos, edge_index):
    pos8 = jnp.pad(pos, ((0, 0), (0, 5)))
    ei_flat = jnp.reshape(edge_index, (2 * M_EDGES,))
    eiout_flat, w, vec = _distance_sc(pos8, ei_flat)
    return (jnp.reshape(eiout_flat, (2, M_EDGES)), w, vec)


# speedup vs baseline: 1.0515x; 1.0515x over previous
import jax
import jax.numpy as jnp
from jax import lax
from jax.experimental import pallas as pl
from jax.experimental.pallas import tpu as pltpu
from jax.experimental.pallas import tpu_sc as plsc

N_NODES = 100000
M_EDGES = 6400000
NUM_WORKERS = 32
CHUNK = 2048
TOTAL_CHUNKS = M_EDGES // CHUNK
BASE_CHUNKS = TOTAL_CHUNKS // NUM_WORKERS
EXTRA = TOTAL_CHUNKS - BASE_CHUNKS * NUM_WORKERS
L = 16


def _newton_sqrt(sq):
    i = plsc.bitcast(sq, jnp.int32)
    i = jnp.int32(0x5F3759DF) - (i >> 1)
    y = plsc.bitcast(i, jnp.float32)
    t = sq * jnp.float32(0.5)
    y = y * (jnp.float32(1.5) - (t * y) * y)
    y = y * (jnp.float32(1.5) - (t * y) * y)
    return sq * y


def _body(pos8_hbm, ei_hbm, eiout_hbm, w_hbm, vec_hbm,
          idx_v, rows_v, vec_v, w_v, sem):
    wid = lax.axis_index("c") * 16 + lax.axis_index("s")
    n_chunks = BASE_CHUNKS + jnp.where(wid < EXTRA, 1, 0)

    iota = lax.iota(jnp.int32, L)
    c0 = jnp.zeros((L,), jnp.int32)
    c1 = jnp.full((L,), 1, jnp.int32)
    c2 = jnp.full((L,), 2, jnp.int32)

    def chunk_body(g, carry):
        t = wid + g * NUM_WORKERS
        base = t * CHUNK
        pltpu.sync_copy(ei_hbm.at[pl.ds(2 * base, 2 * CHUNK)], idx_v)
        pltpu.async_copy(pos8_hbm.at[idx_v], rows_v, sem).wait()

        def compute(i, carry2):
            q0 = (i // 8) * 256 + (i % 8) * 16 + iota
            q1 = q0 + 128
            ep = i * L + iota
            x0 = plsc.load_gather(rows_v, [q0, c0])
            y0 = plsc.load_gather(rows_v, [q0, c1])
            z0 = plsc.load_gather(rows_v, [q0, c2])
            x1 = plsc.load_gather(rows_v, [q1, c0])
            y1 = plsc.load_gather(rows_v, [q1, c1])
            z1 = plsc.load_gather(rows_v, [q1, c2])
            dx = x0 - x1
            dy = y0 - y1
            dz = z0 - z1
            sq = dx * dx + dy * dy + dz * dz
            w = jnp.maximum(_newton_sqrt(sq), jnp.float32(1e-8))
            w_v[pl.ds(i * L, L)] = w
            plsc.store_scatter(vec_v, [ep, c0], dx)
            plsc.store_scatter(vec_v, [ep, c1], dy)
            plsc.store_scatter(vec_v, [ep, c2], dz)
            return carry2

        lax.fori_loop(0, CHUNK // L, compute, 0, unroll=False)

        pltpu.sync_copy(idx_v, eiout_hbm.at[pl.ds(2 * base, 2 * CHUNK)])
        pltpu.sync_copy(w_v, w_hbm.at[pl.ds(base, CHUNK)])
        pltpu.sync_copy(vec_v, vec_hbm.at[pl.ds(base, CHUNK)])
        return carry

    lax.fori_loop(0, n_chunks, chunk_body, 0, unroll=False)


def _distance_sc(pos8, ei_phys):
    mesh = plsc.VectorSubcoreMesh(core_axis_name="c", subcore_axis_name="s")
    k = pl.kernel(
        _body,
        out_type=(
            jax.ShapeDtypeStruct((2 * M_EDGES,), jnp.int32),
            jax.ShapeDtypeStruct((M_EDGES,), jnp.float32),
            jax.ShapeDtypeStruct((M_EDGES, 3), jnp.float32),
        ),
        mesh=mesh,
        compiler_params=pltpu.CompilerParams(use_tc_tiling_on_sc=False,
                                             needs_layout_passes=False),
        scratch_types=[
            pltpu.VMEM((2 * CHUNK,), jnp.int32),
            pltpu.VMEM((2 * CHUNK, 8), jnp.float32),
            pltpu.VMEM((CHUNK, 3), jnp.float32),
            pltpu.VMEM((CHUNK,), jnp.float32),
            pltpu.SemaphoreType.DMA,
        ],
    )
    return k(pos8, ei_phys)


def kernel(pos, edge_index):
    pos8 = jnp.pad(pos, ((0, 0), (0, 5)))
    nb = M_EDGES // 128
    ei_phys = jnp.reshape(
        jnp.transpose(jnp.reshape(edge_index, (2, nb, 128)), (1, 0, 2)),
        (2 * M_EDGES,))
    eiout_phys, w, vec = _distance_sc(pos8, ei_phys)
    eiout = jnp.reshape(
        jnp.transpose(jnp.reshape(eiout_phys, (nb, 2, 128)), (1, 0, 2)),
        (2, M_EDGES))
    return (eiout, w, vec)
